# baseline (speedup 1.0000x reference)
import functools

import jax
import jax.numpy as jnp
from jax import lax
from jax.experimental import pallas as pl
from jax.experimental.pallas import tpu as pltpu
from jax.experimental.pallas import tpu_sc as plsc

B = 16384
F = 64
NC, NS = 2, 16
NW = NC * NS
B_PER_W = B // NW
CHUNK = 128
NCHUNK = B_PER_W // CHUNK


@functools.lru_cache(maxsize=None)
def _make_sc_gather():
    mesh = plsc.VectorSubcoreMesh(core_axis_name="c", subcore_axis_name="s",
                                  num_cores=NC, num_subcores=NS)

    @functools.partial(
        pl.kernel,
        mesh=mesh,
        out_type=(
            jax.ShapeDtypeStruct((B, 2 * F), jnp.float32),
            jax.ShapeDtypeStruct((B, 2 * F), jnp.float32),
        ),
        scratch_types=[
            pltpu.VMEM((B_PER_W,), jnp.int32),
            pltpu.VMEM((B_PER_W,), jnp.int32),
            pltpu.VMEM((2, CHUNK, 2 * F), jnp.float32),
            pltpu.VMEM((2, CHUNK, 2 * F), jnp.float32),
            pltpu.SemaphoreType.DMA,
            pltpu.SemaphoreType.DMA,
        ],
    )
    def _sc_gather(uid_hbm, mid_hbm, ut_hbm, mt_hbm, u_out, m_out,
                   uidx_v, midx_v, urows_v, mrows_v, sem_u, sem_m):
        wid = lax.axis_index("s") * NC + lax.axis_index("c")
        base = wid * B_PER_W
        pltpu.sync_copy(uid_hbm.at[pl.ds(base, B_PER_W)], uidx_v)
        pltpu.sync_copy(mid_hbm.at[pl.ds(base, B_PER_W)], midx_v)
        copies = [None, None]
        for c in range(NCHUNK):
            s = c & 1
            cu = pltpu.async_copy(
                ut_hbm.at[uidx_v.at[pl.ds(c * CHUNK, CHUNK)]],
                urows_v.at[s], sem_u)
            cm = pltpu.async_copy(
                mt_hbm.at[midx_v.at[pl.ds(c * CHUNK, CHUNK)]],
                mrows_v.at[s], sem_m)
            if copies[1 - s] is not None:
                pcu, pcm, pbase = copies[1 - s]
                pcu.wait()
                pcm.wait()
                pltpu.sync_copy(urows_v.at[1 - s], u_out.at[pl.ds(pbase, CHUNK)])
                pltpu.sync_copy(mrows_v.at[1 - s], m_out.at[pl.ds(pbase, CHUNK)])
            copies[s] = (cu, cm, base + c * CHUNK)
        s = (NCHUNK - 1) & 1
        cu, cm, pbase = copies[s]
        cu.wait()
        cm.wait()
        pltpu.sync_copy(urows_v.at[s], u_out.at[pl.ds(pbase, CHUNK)])
        pltpu.sync_copy(mrows_v.at[s], m_out.at[pl.ds(pbase, CHUNK)])

    return _sc_gather


def _mlp_body(u2_ref, m2_ref, up_ref, mp_ref, w1u_ref, w1m_ref, b1_ref,
              w2_ref, b2_ref, w3_ref, b3_ref, o_ref):
    u = jnp.where(up_ref[...] == 1, u2_ref[:, F:], u2_ref[:, :F])
    m = jnp.where(mp_ref[...] == 1, m2_ref[:, F:], m2_ref[:, :F])
    x = u @ w1u_ref[...] + m @ w1m_ref[...] + b1_ref[...]
    x = jnp.maximum(x, 0.0)
    x = jnp.maximum(x @ w2_ref[...] + b2_ref[...], 0.0)
    o_ref[...] = x @ w3_ref[...] + b3_ref[...]


def _mlp(u2, m2, up, mp, w1u, w1m, b1, w2, b2, w3, b3):
    blk = 2048
    grid = (B // blk,)
    full = lambda shape: pl.BlockSpec(shape, lambda i: (0, 0))
    return pl.pallas_call(
        _mlp_body,
        grid=grid,
        in_specs=[
            pl.BlockSpec((blk, 2 * F), lambda i: (i, 0)),
            pl.BlockSpec((blk, 2 * F), lambda i: (i, 0)),
            pl.BlockSpec((blk, 1), lambda i: (i, 0)),
            pl.BlockSpec((blk, 1), lambda i: (i, 0)),
            full(w1u.shape),
            full(w1m.shape),
            full(b1.shape),
            full(w2.shape),
            full(b2.shape),
            full(w3.shape),
            full(b3.shape),
        ],
        out_specs=pl.BlockSpec((blk, 1), lambda i: (i, 0)),
        out_shape=jax.ShapeDtypeStruct((B, 1), jnp.float32),
    )(u2, m2, up, mp, w1u, w1m, b1, w2, b2, w3, b3)


def kernel(user_ids, movie_ids, user_table, movie_table, W1, b1, W2, b2, W3, b3):
    uid = user_ids.astype(jnp.int32)
    mid = movie_ids.astype(jnp.int32)
    ut2 = user_table.reshape(-1, 2 * F)
    mt2 = movie_table.reshape(-1, 2 * F)
    u2, m2 = _make_sc_gather()(uid >> 1, mid >> 1, ut2, mt2)
    up = (uid & 1).reshape(B, 1)
    mp = (mid & 1).reshape(B, 1)
    out = _mlp(u2, m2, up, mp,
               W1[:F], W1[F:], b1.reshape(1, -1),
               W2, b2.reshape(1, -1),
               W3, b3.reshape(1, 1))
    return out.reshape(B)


# speedup vs baseline: 1.6521x; 1.6521x over previous
import functools

import jax
import jax.numpy as jnp
from jax import lax
from jax.experimental import pallas as pl
from jax.experimental.pallas import tpu as pltpu
from jax.experimental.pallas import tpu_sc as plsc

B = 16384
F = 64
NC, NS = 2, 16
NW = NC * NS
B_PER_W = B // NW

import dataclasses
_cp = pltpu.CompilerParams()
if "needs_layout_passes" in pltpu.CompilerParams.__dataclass_fields__:
    _cp = dataclasses.replace(_cp, needs_layout_passes=False)


@functools.lru_cache(maxsize=None)
def _make_sc_gather():
    mesh = plsc.VectorSubcoreMesh(core_axis_name="c", subcore_axis_name="s",
                                  num_cores=NC, num_subcores=NS)

    @functools.partial(
        pl.kernel,
        mesh=mesh,
        compiler_params=_cp,
        out_type=(
            jax.ShapeDtypeStruct((B, F), jnp.float32),
            jax.ShapeDtypeStruct((B, F), jnp.float32),
        ),
        scratch_types=[
            pltpu.VMEM((B_PER_W + 16,), jnp.int32),
            pltpu.VMEM((B_PER_W + 16,), jnp.int32),
            pltpu.VMEM((B_PER_W // 2, F), jnp.float32),
            pltpu.VMEM((B_PER_W // 2, F), jnp.float32),
            pltpu.SemaphoreType.DMA,
            pltpu.SemaphoreType.DMA,
        ],
    )
    def _sc_gather(uid_hbm, mid_hbm, ut_hbm, mt_hbm, u_out, m_out,
                   uidx_v, midx_v, urows_v, mrows_v,
                   sem_u, sem_m):
        wid = lax.axis_index("s") * NC + lax.axis_index("c")
        base = wid * B_PER_W
        half = B_PER_W // 2
        pltpu.sync_copy(uid_hbm.at[pl.ds(base, B_PER_W)],
                        uidx_v.at[pl.ds(0, B_PER_W)])
        pltpu.sync_copy(mid_hbm.at[pl.ds(base, B_PER_W)],
                        midx_v.at[pl.ds(0, B_PER_W)])
        for c in range(2):
            @pl.loop(0, half)
            def _(i):
                ui = uidx_v[pl.ds(c * half + i, 16)][0]
                mi = midx_v[pl.ds(c * half + i, 16)][0]
                pltpu.make_async_copy(ut_hbm.at[pl.ds(ui, 1)],
                                      urows_v.at[pl.ds(i, 1)], sem_u).start()
                pltpu.make_async_copy(mt_hbm.at[pl.ds(mi, 1)],
                                      mrows_v.at[pl.ds(i, 1)], sem_m).start()

            pltpu.make_async_copy(ut_hbm.at[pl.ds(0, half)], urows_v, sem_u).wait()
            pltpu.make_async_copy(mt_hbm.at[pl.ds(0, half)], mrows_v, sem_m).wait()
            pltpu.sync_copy(urows_v, u_out.at[pl.ds(base + c * half, half)])
            pltpu.sync_copy(mrows_v, m_out.at[pl.ds(base + c * half, half)])

    return _sc_gather


def _mlp_body(u_ref, m_ref, w1u_ref, w1m_ref, b1_ref, w2_ref, b2_ref,
              w3_ref, b3_ref, o_ref):
    x = u_ref[...] @ w1u_ref[...] + m_ref[...] @ w1m_ref[...] + b1_ref[...]
    x = jnp.maximum(x, 0.0)
    x = jnp.maximum(x @ w2_ref[...] + b2_ref[...], 0.0)
    o_ref[...] = x @ w3_ref[...] + b3_ref[...]


def _mlp(u, m, w1u, w1m, b1, w2, b2, w3, b3):
    blk = 2048
    grid = (B // blk,)
    full = lambda shape: pl.BlockSpec(shape, lambda i: (0, 0))
    return pl.pallas_call(
        _mlp_body,
        grid=grid,
        in_specs=[
            pl.BlockSpec((blk, F), lambda i: (i, 0)),
            pl.BlockSpec((blk, F), lambda i: (i, 0)),
            full(w1u.shape),
            full(w1m.shape),
            full(b1.shape),
            full(w2.shape),
            full(b2.shape),
            full(w3.shape),
            full(b3.shape),
        ],
        out_specs=pl.BlockSpec((blk, 1), lambda i: (i, 0)),
        out_shape=jax.ShapeDtypeStruct((B, 1), jnp.float32),
    )(u, m, w1u, w1m, b1, w2, b2, w3, b3)


def kernel(user_ids, movie_ids, user_table, movie_table, W1, b1, W2, b2, W3, b3):
    u, m = _make_sc_gather()(user_ids.astype(jnp.int32),
                             movie_ids.astype(jnp.int32),
                             user_table, movie_table)
    out = _mlp(u, m,
               W1[:F], W1[F:], b1.reshape(1, -1),
               W2, b2.reshape(1, -1),
               W3, b3.reshape(1, 1))
    return out.reshape(B)
